# baseline (speedup 1.0000x reference)
import functools

import jax
import jax.numpy as jnp
from jax import lax
from jax.experimental import pallas as pl
from jax.experimental.pallas import tpu as pltpu
from jax.experimental.pallas import tpu_sc as plsc

_NUM_CORES = 2
_NUM_SUBCORES = 16
_NW = _NUM_CORES * _NUM_SUBCORES
_LANES = 16
_R = 8


@functools.lru_cache(maxsize=None)
def _build(B, T, DIM):
    rows_w = T // _NW
    R = _R if rows_w % _R == 0 else rows_w
    n_sub = rows_w // R
    n_tiles = n_sub * B
    groups_row = DIM // _LANES

    mesh = plsc.VectorSubcoreMesh(core_axis_name="c", subcore_axis_name="s")

    @functools.partial(
        pl.kernel,
        out_type=jax.ShapeDtypeStruct((B, T, DIM), jnp.float32),
        mesh=mesh,
        compiler_params=pltpu.CompilerParams(use_tc_tiling_on_sc=True),
        scratch_types=(
            [pltpu.VMEM((R, DIM), jnp.float32) for _ in range(8)]
            + [pltpu.VMEM((R, DIM), jnp.float32) for _ in range(2)]
            + [pltpu.SemaphoreType.DMA for _ in range(18)]
        ),
    )
    def k(x_hbm, pos_hbm, out_hbm,
          xv0, xv1, xv2, xv3, xv4, xv5, xv6, xv7, pv0, pv1,
          sxi0, sxi1, sxi2, sxi3, sxi4, sxi5, sxi6, sxi7,
          soo0, soo1, soo2, soo3, soo4, soo5, soo6, soo7, spi0, spi1):
        wid = lax.axis_index("s") * _NUM_CORES + lax.axis_index("c")
        base = wid * rows_w
        xv = (xv0, xv1, xv2, xv3, xv4, xv5, xv6, xv7)
        pv = (pv0, pv1)
        sxi = (sxi0, sxi1, sxi2, sxi3, sxi4, sxi5, sxi6, sxi7)
        soo = (soo0, soo1, soo2, soo3, soo4, soo5, soo6, soo7)
        spi = (spi0, spi1)

        def x_loc(kk):
            s, b = divmod(kk, B)
            return b, base + s * R

        def start_xin(kk):
            b, r0 = x_loc(kk)
            return pltpu.async_copy(
                x_hbm.at[b, pl.ds(r0, R), :], xv[kk % 8], sxi[kk % 8])

        def start_pin(s):
            return pltpu.async_copy(
                pos_hbm.at[pl.ds(base + s * R, R), :], pv[s % 2], spi[s % 2])

        def start_out(kk):
            b, r0 = x_loc(kk)
            return pltpu.async_copy(
                xv[kk % 8], out_hbm.at[b, pl.ds(r0, R), :], soo[kk % 8])

        pending = {}
        pending["p0"] = start_pin(0)
        for j in range(min(8, n_tiles)):
            pending[f"x{j}"] = start_xin(j)

        for st in range(n_tiles // B):
            ks = [B * st + j for j in range(B)]
            s = st
            pending.pop(f"p{s}").wait()
            if s + 1 < n_sub:
                pending[f"p{s + 1}"] = start_pin(s + 1)
            if st >= 1 and ks[0] + B < n_tiles:
                for j in range(B):
                    pending.pop(f"o{ks[j] - B}").wait()
                    pending[f"x{ks[j] + B}"] = start_xin(ks[j] + B)
            for j in range(B):
                pending.pop(f"x{ks[j]}").wait()

            xb0, xb1, xb2, xb3 = (xv[kkj % 8] for kkj in ks)
            pbuf = pv[s % 2]

            @plsc.parallel_loop(0, R * groups_row, step=1, unroll=8)
            def add_body(i):
                r = i // groups_row
                sl = pl.ds((i % groups_row) * _LANES, _LANES)
                pval = pbuf[r, sl]
                plsc.addupdate(xb0.at[r, sl], pval)
                plsc.addupdate(xb1.at[r, sl], pval)
                plsc.addupdate(xb2.at[r, sl], pval)
                plsc.addupdate(xb3.at[r, sl], pval)

            for j in range(B):
                pending[f"o{ks[j]}"] = start_out(ks[j])

        for h in pending.values():
            h.wait()

    return k


def kernel(x, pos):
    B, T, DIM = x.shape
    return _build(B, T, DIM)(x, pos[:T])


# speedup vs baseline: 1.0072x; 1.0072x over previous
import functools

import jax
import jax.numpy as jnp
from jax import lax
from jax.experimental import pallas as pl
from jax.experimental.pallas import tpu as pltpu
from jax.experimental.pallas import tpu_sc as plsc

_NUM_CORES = 2
_NUM_SUBCORES = 16
_NW = _NUM_CORES * _NUM_SUBCORES
_LANES = 16
_R = 16


@functools.lru_cache(maxsize=None)
def _build(B, T, DIM):
    rows_w = T // _NW
    R = _R if rows_w % _R == 0 else rows_w
    n_sub = rows_w // R
    n_tiles = n_sub * B
    groups_row = DIM // _LANES

    mesh = plsc.VectorSubcoreMesh(core_axis_name="c", subcore_axis_name="s")

    @functools.partial(
        pl.kernel,
        out_type=jax.ShapeDtypeStruct((B, T, DIM), jnp.float32),
        mesh=mesh,
        compiler_params=pltpu.CompilerParams(use_tc_tiling_on_sc=True),
        scratch_types=(
            [pltpu.VMEM((R, DIM), jnp.float32) for _ in range(4)]
            + [pltpu.VMEM((R, DIM), jnp.float32) for _ in range(2)]
            + [pltpu.SemaphoreType.DMA for _ in range(10)]
        ),
    )
    def k(x_hbm, pos_hbm, out_hbm,
          xv0, xv1, xv2, xv3, pv0, pv1,
          sxi0, sxi1, sxi2, sxi3, soo0, soo1, soo2, soo3, spi0, spi1):
        wid = lax.axis_index("s") * _NUM_CORES + lax.axis_index("c")
        base = wid * rows_w
        xv = (xv0, xv1, xv2, xv3)
        pv = (pv0, pv1)
        sxi = (sxi0, sxi1, sxi2, sxi3)
        soo = (soo0, soo1, soo2, soo3)
        spi = (spi0, spi1)

        def x_loc(kk):
            s, b = divmod(kk, B)
            return b, base + s * R

        def start_xin(kk):
            b, r0 = x_loc(kk)
            return pltpu.async_copy(
                x_hbm.at[b, pl.ds(r0, R), :], xv[kk % 4], sxi[kk % 4])

        def start_pin(s):
            return pltpu.async_copy(
                pos_hbm.at[pl.ds(base + s * R, R), :], pv[s % 2], spi[s % 2])

        def start_out_half(kk, h):
            b, r0 = x_loc(kk)
            H = R // 2
            return pltpu.async_copy(
                xv[kk % 4].at[pl.ds(h * H, H), :],
                out_hbm.at[b, pl.ds(r0 + h * H, H), :], soo[kk % 4])

        class _Pair:
            def __init__(self, a, b):
                self._a, self._b = a, b

            def wait(self):
                self._a.wait()
                self._b.wait()

        pending = {}
        pending["p0"] = start_pin(0)
        for j in range(min(4, n_tiles)):
            pending[f"x{j}"] = start_xin(j)

        for st in range(n_tiles // 2):
            kk0 = 2 * st
            kk1 = kk0 + 1
            s, b0 = divmod(kk0, B)
            if b0 == 0:
                pending.pop(f"p{s}").wait()
                if s + 1 < n_sub:
                    pending[f"p{s + 1}"] = start_pin(s + 1)
            if st >= 1 and kk0 + 2 < n_tiles:
                pending.pop(f"o{kk0 - 2}").wait()
                pending[f"x{kk0 + 2}"] = start_xin(kk0 + 2)
                pending.pop(f"o{kk1 - 2}").wait()
                pending[f"x{kk1 + 2}"] = start_xin(kk1 + 2)
            pending.pop(f"x{kk0}").wait()
            pending.pop(f"x{kk1}").wait()

            xb0 = xv[kk0 % 4]
            xb1 = xv[kk1 % 4]
            pbuf = pv[s % 2]

            H = R // 2

            @plsc.parallel_loop(0, H * groups_row, step=1, unroll=8)
            def add_body_lo(i):
                r = i // groups_row
                sl = pl.ds((i % groups_row) * _LANES, _LANES)
                pval = pbuf[r, sl]
                plsc.addupdate(xb0.at[r, sl], pval)
                plsc.addupdate(xb1.at[r, sl], pval)

            oa0 = start_out_half(kk0, 0)
            oa1 = start_out_half(kk1, 0)

            @plsc.parallel_loop(H * groups_row, R * groups_row, step=1,
                                unroll=8)
            def add_body_hi(i):
                r = i // groups_row
                sl = pl.ds((i % groups_row) * _LANES, _LANES)
                pval = pbuf[r, sl]
                plsc.addupdate(xb0.at[r, sl], pval)
                plsc.addupdate(xb1.at[r, sl], pval)

            ob0 = start_out_half(kk0, 1)
            ob1 = start_out_half(kk1, 1)
            pending[f"o{kk0}"] = _Pair(oa0, ob0)
            pending[f"o{kk1}"] = _Pair(oa1, ob1)

        for h in pending.values():
            h.wait()

    return k


def kernel(x, pos):
    B, T, DIM = x.shape
    return _build(B, T, DIM)(x, pos[:T])


# speedup vs baseline: 1.0353x; 1.0280x over previous
import functools

import jax
import jax.numpy as jnp
from jax import lax
from jax.experimental import pallas as pl
from jax.experimental.pallas import tpu as pltpu
from jax.experimental.pallas import tpu_sc as plsc

_NUM_CORES = 2
_NUM_SUBCORES = 16
_NW = _NUM_CORES * _NUM_SUBCORES
_LANES = 16
_R = 16


@functools.lru_cache(maxsize=None)
def _build(B, T, DIM):
    rows_w = T // _NW
    R = _R if rows_w % _R == 0 else rows_w
    n_sub = rows_w // R
    n_tiles = n_sub * B
    groups_row = DIM // _LANES

    mesh = plsc.VectorSubcoreMesh(core_axis_name="c", subcore_axis_name="s")

    @functools.partial(
        pl.kernel,
        out_type=jax.ShapeDtypeStruct((B, T, DIM), jnp.float32),
        mesh=mesh,
        compiler_params=pltpu.CompilerParams(use_tc_tiling_on_sc=True),
        scratch_types=(
            [pltpu.VMEM((R, DIM), jnp.float32) for _ in range(4)]
            + [pltpu.VMEM((R, DIM), jnp.float32) for _ in range(2)]
            + [pltpu.SemaphoreType.DMA for _ in range(10)]
        ),
    )
    def k(x_hbm, pos_hbm, out_hbm,
          xv0, xv1, xv2, xv3, pv0, pv1,
          sxi0, sxi1, sxi2, sxi3, soo0, soo1, soo2, soo3, spi0, spi1):
        wid = lax.axis_index("s") * _NUM_CORES + lax.axis_index("c")
        base = wid * rows_w
        xv = (xv0, xv1, xv2, xv3)
        pv = (pv0, pv1)
        sxi = (sxi0, sxi1, sxi2, sxi3)
        soo = (soo0, soo1, soo2, soo3)
        spi = (spi0, spi1)

        def x_loc(kk):
            s, b = divmod(kk, B)
            return b, base + s * R

        def start_xin(kk):
            b, r0 = x_loc(kk)
            return pltpu.async_copy(
                x_hbm.at[b, pl.ds(r0, R), :], xv[kk % 4], sxi[kk % 4])

        def start_pin(s):
            return pltpu.async_copy(
                pos_hbm.at[pl.ds(base + s * R, R), :], pv[s % 2], spi[s % 2])

        def start_out(kk):
            b, r0 = x_loc(kk)
            return pltpu.async_copy(
                xv[kk % 4], out_hbm.at[b, pl.ds(r0, R), :], soo[kk % 4])

        pending = {}
        pending["p0"] = start_pin(0)
        for j in range(min(4, n_tiles)):
            pending[f"x{j}"] = start_xin(j)

        for st in range(n_tiles // 2):
            kk0 = 2 * st
            kk1 = kk0 + 1
            s, b0 = divmod(kk0, B)
            if b0 == 0:
                pending.pop(f"p{s}").wait()
                if s + 1 < n_sub:
                    pending[f"p{s + 1}"] = start_pin(s + 1)
            if st >= 1 and kk0 + 2 < n_tiles:
                pending.pop(f"o{kk0 - 2}").wait()
                pending[f"x{kk0 + 2}"] = start_xin(kk0 + 2)
                pending.pop(f"o{kk1 - 2}").wait()
                pending[f"x{kk1 + 2}"] = start_xin(kk1 + 2)
            pending.pop(f"x{kk0}").wait()
            pending.pop(f"x{kk1}").wait()

            xb0 = xv[kk0 % 4]
            xb1 = xv[kk1 % 4]
            pbuf = pv[s % 2]

            @plsc.parallel_loop(0, R * groups_row, step=1, unroll=8)
            def add_body(i):
                r = i // groups_row
                sl = pl.ds((i % groups_row) * _LANES, _LANES)
                pval = pbuf[r, sl]
                plsc.addupdate(xb0.at[r, sl], pval)
                plsc.addupdate(xb1.at[r, sl], pval)

            pending[f"o{kk0}"] = start_out(kk0)
            pending[f"o{kk1}"] = start_out(kk1)

        for h in pending.values():
            h.wait()

    return k


def kernel(x, pos):
    B, T, DIM = x.shape
    return _build(B, T, DIM)(x, pos[:T])
